# baseline (speedup 1.0000x reference)
import jax
import jax.numpy as jnp
from jax import lax
from jax.experimental import pallas as pl
from jax.experimental.pallas import tpu as pltpu
from jax.experimental.pallas import tpu_sc as plsc

_B = 4096
_L = 200
_D = 64
_TOT = _B * _L
_NW = 32
_PER_W = _TOT // _NW
_CHUNK = 512
_KSUB = _CHUNK // 128
_NCHUNK = _PER_W // _CHUNK
_IROWS_W = _PER_W // 128


def _body(idx_hbm, tab_hbm, out_hbm, idx_v, rows_v, gsem):
    wid = lax.axis_index("s") * 2 + lax.axis_index("c")
    row0 = wid * _IROWS_W

    lanes = lax.iota(jnp.int32, 16)
    zeros16 = jnp.zeros((16,), jnp.float32)

    def step(ci, carry):
        irow = row0 + ci * _KSUB
        base = row0 * 128 + ci * _CHUNK
        pltpu.sync_copy(idx_hbm.at[pl.ds(irow, _KSUB)], idx_v)
        copies = [
            pltpu.async_copy(
                tab_hbm.at[idx_v.at[j]],
                rows_v.at[pl.ds(j * 128, 128)],
                gsem,
            )
            for j in range(_KSUB)
        ]
        for c in copies:
            c.wait()

        for r in range(_KSUB):
            def mask_group(g, carry2, r=r):
                idxv = idx_v[r, pl.ds(g * 16, 16)]
                m = idxv == 0
                rowi = (r * 128 + g * 16) + lanes
                coli = jnp.zeros((16,), jnp.int32)
                for _ in range(_D):
                    plsc.store_scatter(rows_v, [rowi, coli], zeros16,
                                       mask=m)
                    coli = coli + 1
                return carry2

            lax.fori_loop(0, 8, mask_group, None)

        pltpu.sync_copy(rows_v, out_hbm.at[pl.ds(base, _CHUNK)])
        return carry

    lax.fori_loop(0, _NCHUNK, step, None)


def kernel(inputs, embeddings):
    idx = inputs.reshape(_TOT).astype(jnp.int32).reshape(_TOT // 128, 128)
    mesh = plsc.VectorSubcoreMesh(core_axis_name="c", subcore_axis_name="s")
    out = pl.kernel(
        _body,
        mesh=mesh,
        compiler_params=pltpu.CompilerParams(
            needs_layout_passes=False, use_tc_tiling_on_sc=False
        ),
        out_type=jax.ShapeDtypeStruct((_TOT, _D), jnp.float32),
        scratch_types=[
            pltpu.VMEM((_KSUB, 128), jnp.int32),
            pltpu.VMEM((_CHUNK, _D), jnp.float32),
            pltpu.SemaphoreType.DMA,
        ],
    )(idx, embeddings)
    return out.reshape(_B, _L, _D)
